# baseline (speedup 1.0000x reference)
import functools

import jax
import jax.numpy as jnp
from jax import lax
from jax.experimental import pallas as pl
from jax.experimental.pallas import tpu as pltpu
from jax.experimental.pallas import tpu_sc as plsc

N = 10000
NPAD = 10240
D = 128
NC = 2
NS = 16
NW = NC * NS
CHUNK = 112
NBUF = 3
NIDX = 6
ROWS_PER_TILE = NPAD // NS


def _sc_spmm(nchunk0, nchunk1):
    mesh = plsc.VectorSubcoreMesh(core_axis_name="c", subcore_axis_name="s")

    @functools.partial(
        pl.kernel,
        out_type=jax.ShapeDtypeStruct((NC, NPAD, D), jnp.float32),
        mesh=mesh,
        compiler_params=pltpu.CompilerParams(use_tc_tiling_on_sc=False),
        scratch_types=[
            [pltpu.VMEM((CHUNK,), jnp.int32)] * NIDX,
            [pltpu.VMEM((CHUNK,), jnp.int32)] * NIDX,
            [pltpu.VMEM((CHUNK,), jnp.float32)] * NIDX,
            [pltpu.VMEM((CHUNK, D), jnp.float32)] * NBUF,
            pltpu.VMEM_SHARED((NPAD, D), jnp.float32),
            [pltpu.SemaphoreType.DMA] * NIDX,
            [pltpu.SemaphoreType.DMA] * NIDX,
            [pltpu.SemaphoreType.DMA] * NIDX,
            [pltpu.SemaphoreType.DMA] * NBUF,
            [pltpu.SemaphoreType.DMA] * NBUF,
        ],
    )
    def k(x_hbm, cols_hbm, rows_hbm, vals_hbm, zeros_hbm, z_hbm,
          col_slots, row_slots, val_slots, bufs, acc,
          c_sems, r_sems, v_sems, g_sems, s_sems):
        cid = lax.axis_index("c")
        sid = lax.axis_index("s")
        nch = jnp.where(cid == 0, nchunk0, nchunk1)
        base = cid * (NS * nchunk0 * CHUNK) + sid * (nch * CHUNK)

        with jax.named_scope("acc_init"):
            pltpu.sync_copy(zeros_hbm,
                            acc.at[pl.ds(sid * ROWS_PER_TILE, ROWS_PER_TILE)])

        def _esl(j):
            return pl.ds(pl.multiple_of(base + j * CHUNK, 8), CHUNK)

        def load_idx(j, q):
            pltpu.async_copy(cols_hbm.at[_esl(j)], col_slots[q], c_sems[q])
            pltpu.async_copy(rows_hbm.at[_esl(j)], row_slots[q], r_sems[q])
            pltpu.async_copy(vals_hbm.at[_esl(j)], val_slots[q], v_sems[q])

        def wait_idx(j, q):
            pltpu.make_async_copy(cols_hbm.at[_esl(j)], col_slots[q],
                                  c_sems[q]).wait()
            pltpu.make_async_copy(rows_hbm.at[_esl(j)], row_slots[q],
                                  r_sems[q]).wait()
            pltpu.make_async_copy(vals_hbm.at[_esl(j)], val_slots[q],
                                  v_sems[q]).wait()

        def gather(b, q):
            pltpu.async_copy(x_hbm.at[col_slots[q]], bufs[b], g_sems[b])

        def wait_gather(b, q):
            pltpu.make_async_copy(x_hbm.at[col_slots[q]], bufs[b],
                                  g_sems[b]).wait()

        def scatter(b, q):
            pltpu.async_copy(bufs[b], acc.at[row_slots[q]],
                             s_sems[b], add=True)

        def wait_scatter(b, q):
            pltpu.make_async_copy(bufs[b], acc.at[row_slots[q]],
                                  s_sems[b]).wait()

        def scale(b, q):
            @pl.loop(0, CHUNK // 16)
            def _scale(bgrp):
                v_vec = val_slots[q][pl.ds(bgrp * 16, 16)]
                for i in range(16):
                    v = v_vec[i]
                    e = bgrp * 16 + i
                    for kk in range(D // 16):
                        sl = pl.ds(kk * 16, 16)
                        bufs[b][e, sl] = bufs[b][e, sl] * v

        with jax.named_scope("prime"):
            for q in range(3):
                load_idx(q, q)
            plsc.subcore_barrier()
            for b in range(2):
                wait_idx(b, b)
                gather(b, b)

        loop_scope = jax.named_scope("edge_loop")
        loop_scope.__enter__()

        @pl.loop(0, nch // NIDX)
        def _grp(p):
            for c in range(NIDX):
                j = NIDX * p + c
                b = c % NBUF
                wait_gather(b, c)
                scale(b, c)
                scatter(b, c)

                @pl.when(j >= 1)
                def _():
                    wait_scatter((b + NBUF - 1) % NBUF, (c + NIDX - 1) % NIDX)

                @pl.when(j + 3 < nch)
                def _():
                    load_idx(j + 3, (c + 3) % NIDX)

                @pl.when(j + 2 < nch)
                def _():
                    wait_idx(j + 2, (c + 2) % NIDX)
                    gather((b + 2) % NBUF, (c + 2) % NIDX)

        wait_scatter(2, 5)
        loop_scope.__exit__(None, None, None)

        with jax.named_scope("copy_out"):
            plsc.subcore_barrier()
            pltpu.sync_copy(
                acc.at[pl.ds(sid * ROWS_PER_TILE, ROWS_PER_TILE)],
                z_hbm.at[cid, pl.ds(sid * ROWS_PER_TILE, ROWS_PER_TILE)])

    return k


def _tc_linear_body(z_ref, wt_ref, b_ref, o_ref):
    zsum = z_ref[0] + z_ref[1]
    o_ref[...] = (
        jnp.dot(zsum, wt_ref[...], preferred_element_type=jnp.float32)
        + b_ref[...]
    )


def _tc_linear(z, wt, b2d):
    rows_blk = 1000
    return pl.pallas_call(
        _tc_linear_body,
        grid=(N // rows_blk,),
        in_specs=[
            pl.BlockSpec((NC, rows_blk, D), lambda i: (0, i, 0)),
            pl.BlockSpec((D, D), lambda i: (0, 0)),
            pl.BlockSpec((1, D), lambda i: (0, 0)),
        ],
        out_specs=pl.BlockSpec((rows_blk, D), lambda i: (i, 0)),
        out_shape=jax.ShapeDtypeStruct((N, D), jnp.float32),
    )(z, wt, b2d)


def kernel(x, rows, cols, vals, W, b):
    nnz = rows.shape[0]
    grain = NW * NIDX * CHUNK
    total = -(-nnz // grain) * NIDX * CHUNK * NW
    per_pair = total // NS
    gchunk = NIDX * CHUNK
    per0 = (int(per_pair * 0.70) // gchunk) * gchunk
    per1 = per_pair - per0
    nchunk0, nchunk1 = per0 // CHUNK, per1 // CHUNK
    pad = total - nnz

    rows_i = jnp.pad(rows.astype(jnp.int32), (0, pad))
    cols_i = jnp.pad(cols.astype(jnp.int32), (0, pad))
    vals_f = jnp.pad(vals, (0, pad))
    zeros = jnp.zeros((ROWS_PER_TILE, D), jnp.float32)

    z = _sc_spmm(nchunk0, nchunk1)(x, cols_i, rows_i, vals_f, zeros)
    return _tc_linear(z, W.T, b.reshape(1, D))


# speedup vs baseline: 1.0133x; 1.0133x over previous
import functools

import jax
import jax.numpy as jnp
from jax import lax
from jax.experimental import pallas as pl
from jax.experimental.pallas import tpu as pltpu
from jax.experimental.pallas import tpu_sc as plsc

N = 10000
NPAD = 10240
D = 128
NC = 2
NS = 16
NW = NC * NS
CHUNK = 112
NBUF = 3
NIDX = 6
ROWS_PER_TILE = NPAD // NS


def _sc_spmm(nchunk0, nchunk1):
    mesh = plsc.VectorSubcoreMesh(core_axis_name="c", subcore_axis_name="s")

    @functools.partial(
        pl.kernel,
        out_type=jax.ShapeDtypeStruct((NC, NPAD, D), jnp.float32),
        mesh=mesh,
        compiler_params=pltpu.CompilerParams(use_tc_tiling_on_sc=False),
        scratch_types=[
            [pltpu.VMEM((CHUNK,), jnp.int32)] * NIDX,
            [pltpu.VMEM((CHUNK,), jnp.int32)] * NIDX,
            [pltpu.VMEM((CHUNK,), jnp.float32)] * NIDX,
            [pltpu.VMEM((CHUNK, D), jnp.float32)] * NBUF,
            pltpu.VMEM_SHARED((NPAD, D), jnp.float32),
            [pltpu.SemaphoreType.DMA] * NIDX,
            [pltpu.SemaphoreType.DMA] * NIDX,
            [pltpu.SemaphoreType.DMA] * NIDX,
            [pltpu.SemaphoreType.DMA] * NBUF,
            [pltpu.SemaphoreType.DMA] * NBUF,
        ],
    )
    def k(x_hbm, cols_hbm, rows_hbm, vals_hbm, zeros_hbm, z_hbm,
          col_slots, row_slots, val_slots, bufs, acc,
          c_sems, r_sems, v_sems, g_sems, s_sems):
        cid = lax.axis_index("c")
        sid = lax.axis_index("s")
        nch = jnp.where(cid == 0, nchunk0, nchunk1)
        base = cid * (NS * nchunk0 * CHUNK) + sid * (nch * CHUNK)

        with jax.named_scope("acc_init"):
            pltpu.sync_copy(zeros_hbm,
                            acc.at[pl.ds(sid * ROWS_PER_TILE, ROWS_PER_TILE)])

        def _esl(j):
            return pl.ds(pl.multiple_of(base + j * CHUNK, 8), CHUNK)

        def load_idx(j, q):
            pltpu.async_copy(cols_hbm.at[_esl(j)], col_slots[q], c_sems[q])
            pltpu.async_copy(rows_hbm.at[_esl(j)], row_slots[q], r_sems[q])
            pltpu.async_copy(vals_hbm.at[_esl(j)], val_slots[q], v_sems[q])

        def wait_idx(j, q):
            pltpu.make_async_copy(cols_hbm.at[_esl(j)], col_slots[q],
                                  c_sems[q]).wait()
            pltpu.make_async_copy(rows_hbm.at[_esl(j)], row_slots[q],
                                  r_sems[q]).wait()
            pltpu.make_async_copy(vals_hbm.at[_esl(j)], val_slots[q],
                                  v_sems[q]).wait()

        def gather(b, q):
            pltpu.async_copy(x_hbm.at[col_slots[q]], bufs[b], g_sems[b])

        def wait_gather(b, q):
            pltpu.make_async_copy(x_hbm.at[col_slots[q]], bufs[b],
                                  g_sems[b]).wait()

        def scatter(b, q):
            pltpu.async_copy(bufs[b], acc.at[row_slots[q]],
                             s_sems[b], add=True)

        def wait_scatter(b, q):
            pltpu.make_async_copy(bufs[b], acc.at[row_slots[q]],
                                  s_sems[b]).wait()

        def scale(b, q):
            @pl.loop(0, CHUNK // 16)
            def _scale(bgrp):
                v_vec = val_slots[q][pl.ds(bgrp * 16, 16)]
                for i in range(16):
                    v = v_vec[i]
                    e = bgrp * 16 + i
                    for kk in range(D // 16):
                        sl = pl.ds(kk * 16, 16)
                        bufs[b][e, sl] = bufs[b][e, sl] * v

        with jax.named_scope("prime"):
            for q in range(3):
                load_idx(q, q)
            plsc.subcore_barrier()
            for b in range(2):
                wait_idx(b, b)
                gather(b, b)

        loop_scope = jax.named_scope("edge_loop")
        loop_scope.__enter__()

        @pl.loop(0, nch // NIDX)
        def _grp(p):
            for c in range(NIDX):
                j = NIDX * p + c
                b = c % NBUF
                wait_gather(b, c)
                scale(b, c)
                scatter(b, c)

                @pl.when(j >= 1)
                def _():
                    wait_scatter((b + NBUF - 1) % NBUF, (c + NIDX - 1) % NIDX)

                @pl.when(j + 3 < nch)
                def _():
                    load_idx(j + 3, (c + 3) % NIDX)

                @pl.when(j + 2 < nch)
                def _():
                    wait_idx(j + 2, (c + 2) % NIDX)
                    gather((b + 2) % NBUF, (c + 2) % NIDX)

        wait_scatter(2, 5)
        loop_scope.__exit__(None, None, None)

        with jax.named_scope("copy_out"):
            plsc.subcore_barrier()
            pltpu.sync_copy(
                acc.at[pl.ds(sid * ROWS_PER_TILE, ROWS_PER_TILE)],
                z_hbm.at[cid, pl.ds(sid * ROWS_PER_TILE, ROWS_PER_TILE)])

    return k


def _tc_linear_body(z_ref, wt_ref, b_ref, o_ref):
    zsum = z_ref[0] + z_ref[1]
    o_ref[...] = (
        jnp.dot(zsum, wt_ref[...], preferred_element_type=jnp.float32)
        + b_ref[...]
    )


def _tc_linear(z, wt, b2d):
    rows_blk = 1000
    return pl.pallas_call(
        _tc_linear_body,
        grid=(N // rows_blk,),
        in_specs=[
            pl.BlockSpec((NC, rows_blk, D), lambda i: (0, i, 0)),
            pl.BlockSpec((D, D), lambda i: (0, 0)),
            pl.BlockSpec((1, D), lambda i: (0, 0)),
        ],
        out_specs=pl.BlockSpec((rows_blk, D), lambda i: (i, 0)),
        out_shape=jax.ShapeDtypeStruct((N, D), jnp.float32),
    )(z, wt, b2d)


def kernel(x, rows, cols, vals, W, b):
    nnz = rows.shape[0]
    grain = NW * NIDX * CHUNK
    total = -(-nnz // grain) * NIDX * CHUNK * NW
    per_pair = total // NS
    gchunk = NIDX * CHUNK
    per0 = (int(per_pair * 0.75) // gchunk) * gchunk
    per1 = per_pair - per0
    nchunk0, nchunk1 = per0 // CHUNK, per1 // CHUNK
    pad = total - nnz

    rows_i = jnp.pad(rows.astype(jnp.int32), (0, pad))
    cols_i = jnp.pad(cols.astype(jnp.int32), (0, pad))
    vals_f = jnp.pad(vals, (0, pad))
    zeros = jnp.zeros((ROWS_PER_TILE, D), jnp.float32)

    z = _sc_spmm(nchunk0, nchunk1)(x, cols_i, rows_i, vals_f, zeros)
    return _tc_linear(z, W.T, b.reshape(1, D))
